# Initial kernel scaffold; baseline (speedup 1.0000x reference)
import jax
import jax.numpy as jnp
from jax.experimental import pallas as pl


def kernel(points, index, params):
    raise NotImplementedError("write your pallas kernel here")


# speedup vs baseline: 1.0787x; 1.0787x over previous
import functools

import jax
import jax.numpy as jnp
from jax.experimental import pallas as pl

B = 8
T = 50000
DIM = 3
H = 32
CD = 32
RESO = 128
NS = RESO * RESO
NB = 5

ROWS = B * T
BLK = 4000


def _front_body(p_ref, wpos, bpos, w0, b0, w1, b1, ws, net_out):
    x = jnp.dot(p_ref[...], wpos[...], preferred_element_type=jnp.float32) + bpos[...]
    h = jnp.dot(jnp.maximum(x, 0.0), w0[...], preferred_element_type=jnp.float32) + b0[...]
    dx = jnp.dot(jnp.maximum(h, 0.0), w1[...], preferred_element_type=jnp.float32) + b1[...]
    net_out[...] = jnp.dot(x, ws[...], preferred_element_type=jnp.float32) + dx


def _mid_body(net_ref, pool_ref, w0, b0, w1, b1, ws, out_ref):
    n = net_ref[...]
    p = pool_ref[...]
    x = jnp.concatenate([n, p], axis=-1)
    h = jnp.dot(jnp.maximum(x, 0.0), w0[...], preferred_element_type=jnp.float32) + b0[...]
    dx = jnp.dot(jnp.maximum(h, 0.0), w1[...], preferred_element_type=jnp.float32) + b1[...]
    out_ref[...] = jnp.dot(x, ws[...], preferred_element_type=jnp.float32) + dx


def _last_body(net_ref, pool_ref, w0, b0, w1, b1, ws, wc, bc, out_ref):
    n = net_ref[...]
    p = pool_ref[...]
    x = jnp.concatenate([n, p], axis=-1)
    h = jnp.dot(jnp.maximum(x, 0.0), w0[...], preferred_element_type=jnp.float32) + b0[...]
    dx = jnp.dot(jnp.maximum(h, 0.0), w1[...], preferred_element_type=jnp.float32) + b1[...]
    net = jnp.dot(x, ws[...], preferred_element_type=jnp.float32) + dx
    out_ref[...] = jnp.dot(net, wc[...], preferred_element_type=jnp.float32) + bc[...]


def _row_spec(width):
    return pl.BlockSpec((BLK, width), lambda i: (i, 0))


def _full_spec(shape):
    nd = len(shape)
    return pl.BlockSpec(shape, lambda i: (0,) * nd)


def _front(points, params):
    b0p = params['blocks'][0]
    args = (points.reshape(ROWS, DIM), params['Wpos'], params['bpos'].reshape(1, 2 * H),
            b0p['W0'], b0p['b0'].reshape(1, H), b0p['W1'], b0p['b1'].reshape(1, H),
            b0p['Ws'])
    return pl.pallas_call(
        _front_body,
        grid=(ROWS // BLK,),
        in_specs=[_row_spec(DIM)] + [_full_spec(a.shape) for a in args[1:]],
        out_specs=_row_spec(H),
        out_shape=jax.ShapeDtypeStruct((ROWS, H), jnp.float32),
    )(*args)


def _mid(net, pooled, bp):
    args = (net, pooled, bp['W0'], bp['b0'].reshape(1, H), bp['W1'],
            bp['b1'].reshape(1, H), bp['Ws'])
    return pl.pallas_call(
        _mid_body,
        grid=(ROWS // BLK,),
        in_specs=[_row_spec(H), _row_spec(H)] + [_full_spec(a.shape) for a in args[2:]],
        out_specs=_row_spec(H),
        out_shape=jax.ShapeDtypeStruct((ROWS, H), jnp.float32),
    )(*args)


def _last(net, pooled, bp, wc, bc):
    args = (net, pooled, bp['W0'], bp['b0'].reshape(1, H), bp['W1'],
            bp['b1'].reshape(1, H), bp['Ws'], wc, bc.reshape(1, CD))
    return pl.pallas_call(
        _last_body,
        grid=(ROWS // BLK,),
        in_specs=[_row_spec(H), _row_spec(H)] + [_full_spec(a.shape) for a in args[2:]],
        out_specs=_row_spec(CD),
        out_shape=jax.ShapeDtypeStruct((ROWS, CD), jnp.float32),
    )(*args)


def _pool_local(net, idx):
    def one(nb, ib):
        seg = jax.ops.segment_max(nb, ib, num_segments=NS)
        seg = jnp.where(jnp.isfinite(seg), seg, 0.0)
        return jnp.take(seg, ib, axis=0)
    return jax.vmap(one)(net, idx)


def _plane_features(c, idx):
    def one(cb, ib):
        s = jax.ops.segment_sum(cb, ib, num_segments=NS)
        cnt = jax.ops.segment_sum(jnp.ones((cb.shape[0],), jnp.float32), ib, num_segments=NS)
        mean = s / jnp.maximum(cnt, 1.0)[:, None]
        return jnp.transpose(mean).reshape(CD, RESO, RESO)
    return jax.vmap(one)(c, idx)


def kernel(points, index, params):
    idx = index[:, 0, :]
    net = _front(points, params).reshape(B, T, H)
    for i in range(1, NB - 1):
        pooled = _pool_local(net, idx)
        net = _mid(net.reshape(ROWS, H), pooled.reshape(ROWS, H),
                   params['blocks'][i]).reshape(B, T, H)
    pooled = _pool_local(net, idx)
    c = _last(net.reshape(ROWS, H), pooled.reshape(ROWS, H),
              params['blocks'][NB - 1], params['Wc'], params['bc']).reshape(B, T, CD)
    return _plane_features(c, idx)


# speedup vs baseline: 4.1514x; 3.8484x over previous
import functools

import jax
import jax.numpy as jnp
from jax import lax
from jax.experimental import pallas as pl
from jax.experimental.pallas import tpu as pltpu
from jax.experimental.pallas import tpu_sc as plsc

B = 8
T = 50000
DIM = 3
H = 32
CD = 32
RESO = 128
NS = RESO * RESO
NB = 5

ROWS = B * T
BLK = 4000

NE = 8
ESEG = NS // NE
CHUNK = 2000
NCH = T // CHUNK
SLOT = CHUNK + 64

MTS = 3128
MCH = 512
NMC = 6

_SC = pltpu.CompilerParams(needs_layout_passes=False, use_tc_tiling_on_sc=False)
_MESH = plsc.VectorSubcoreMesh(core_axis_name="c", subcore_axis_name="s")


def _front_body(p_ref, wpos, bpos, w0, b0, w1, b1, ws, net_out):
    x = jnp.dot(p_ref[...], wpos[...], preferred_element_type=jnp.float32) + bpos[...]
    h = jnp.dot(jnp.maximum(x, 0.0), w0[...], preferred_element_type=jnp.float32) + b0[...]
    dx = jnp.dot(jnp.maximum(h, 0.0), w1[...], preferred_element_type=jnp.float32) + b1[...]
    net_out[...] = jnp.dot(x, ws[...], preferred_element_type=jnp.float32) + dx


def _mid_body(net_ref, pool_ref, w0, b0, w1, b1, ws, out_ref):
    x = jnp.concatenate([net_ref[...], pool_ref[...]], axis=-1)
    h = jnp.dot(jnp.maximum(x, 0.0), w0[...], preferred_element_type=jnp.float32) + b0[...]
    dx = jnp.dot(jnp.maximum(h, 0.0), w1[...], preferred_element_type=jnp.float32) + b1[...]
    out_ref[...] = jnp.dot(x, ws[...], preferred_element_type=jnp.float32) + dx


def _last_body(net_ref, pool_ref, w0, b0, w1, b1, ws, wc, bc, out_ref):
    x = jnp.concatenate([net_ref[...], pool_ref[...]], axis=-1)
    h = jnp.dot(jnp.maximum(x, 0.0), w0[...], preferred_element_type=jnp.float32) + b0[...]
    dx = jnp.dot(jnp.maximum(h, 0.0), w1[...], preferred_element_type=jnp.float32) + b1[...]
    net = jnp.dot(x, ws[...], preferred_element_type=jnp.float32) + dx
    out_ref[...] = jnp.dot(net, wc[...], preferred_element_type=jnp.float32) + bc[...]


def _tr_body(in_ref, out_ref):
    out_ref[...] = jnp.transpose(in_ref[...], (1, 0))[None]


def _row_spec(width):
    return pl.BlockSpec((BLK, width), lambda i: (i, 0))


def _full_spec(shape):
    nd = len(shape)
    return pl.BlockSpec(shape, lambda i: (0,) * nd)


def _front(points, params):
    bp = params['blocks'][0]
    args = (points.reshape(ROWS, DIM), params['Wpos'], params['bpos'].reshape(1, 2 * H),
            bp['W0'], bp['b0'].reshape(1, H), bp['W1'], bp['b1'].reshape(1, H), bp['Ws'])
    return pl.pallas_call(
        _front_body,
        grid=(ROWS // BLK,),
        in_specs=[_row_spec(DIM)] + [_full_spec(a.shape) for a in args[1:]],
        out_specs=_row_spec(H),
        out_shape=jax.ShapeDtypeStruct((ROWS, H), jnp.float32),
    )(*args)


def _mid(net, pooled, bp):
    args = (net, pooled, bp['W0'], bp['b0'].reshape(1, H), bp['W1'],
            bp['b1'].reshape(1, H), bp['Ws'])
    return pl.pallas_call(
        _mid_body,
        grid=(ROWS // BLK,),
        in_specs=[_row_spec(H), _row_spec(H)] + [_full_spec(a.shape) for a in args[2:]],
        out_specs=_row_spec(H),
        out_shape=jax.ShapeDtypeStruct((ROWS, H), jnp.float32),
    )(*args)


def _last(net, pooled, bp, wc, bc):
    args = (net, pooled, bp['W0'], bp['b0'].reshape(1, H), bp['W1'],
            bp['b1'].reshape(1, H), bp['Ws'], wc, bc.reshape(1, CD))
    return pl.pallas_call(
        _last_body,
        grid=(ROWS // BLK,),
        in_specs=[_row_spec(H), _row_spec(H)] + [_full_spec(a.shape) for a in args[2:]],
        out_specs=_row_spec(CD),
        out_shape=jax.ShapeDtypeStruct((ROWS, CD), jnp.float32),
    )(*args)


def _transpose_mean(mean_flat):
    return pl.pallas_call(
        _tr_body,
        grid=(B, NS // 2048),
        in_specs=[pl.BlockSpec((2048, CD), lambda b, t: (b * (NS // 2048) + t, 0))],
        out_specs=pl.BlockSpec((1, CD, 2048), lambda b, t: (b, 0, t)),
        out_shape=jax.ShapeDtypeStruct((B, CD, NS), jnp.float32),
    )(mean_flat)


def _vsplat(v, lane):
    dn = lax.GatherDimensionNumbers(offset_dims=(), collapsed_slice_dims=(0,),
                                    start_index_map=(0,))
    return lax.gather(v, jnp.full((16, 1), lane, jnp.int32), dn, slice_sizes=(1,),
                      mode=lax.GatherScatterMode.PROMISE_IN_BOUNDS)


@functools.partial(
    pl.kernel,
    out_type=[
        jax.ShapeDtypeStruct((B, NE, NCH, SLOT), jnp.int32),
        jax.ShapeDtypeStruct((B, NE, NCH, SLOT), jnp.int32),
        jax.ShapeDtypeStruct((B, 4, 64), jnp.int32),
    ],
    mesh=_MESH,
    compiler_params=_SC,
    scratch_types=[
        pltpu.VMEM((CHUNK,), jnp.int32),
        pltpu.VMEM((SLOT,), jnp.int32),
        pltpu.VMEM((SLOT,), jnp.int32),
        pltpu.VMEM((SLOT,), jnp.int32),
        pltpu.VMEM((SLOT,), jnp.int32),
        pltpu.VMEM((64,), jnp.int32),
        pltpu.VMEM((16,), jnp.int32),
    ],
)
def _route_kernel(idx_hbm, plist, slist, counts,
                  ibuf, plb0, slb0, plb1, slb1, cbuf, t16):
    cc = lax.axis_index("c")
    ss = lax.axis_index("s")
    b = cc * 4 + ss // 4
    q = ss % 4
    lo = q * (2 * ESEG)
    io = lax.iota(jnp.int32, 16)

    def zb(i, carry):
        cbuf[pl.ds(i * 16, 16)] = jnp.zeros((16,), jnp.int32)
        return carry
    lax.fori_loop(0, 4, zb, 0)

    def emit(plb, slb, m, rel, pid, cnt):
        cum = m.astype(jnp.int32)
        for d in (1, 2, 4, 8):
            t16[...] = cum
            sh = plsc.load_gather(t16, [jnp.maximum(io - d, 0)])
            cum = cum + jnp.where(io >= d, sh, 0)
        pos = cnt + cum - 1
        plsc.store_scatter(plb, [pos], pid, mask=m)
        plsc.store_scatter(slb, [pos], rel, mask=m)
        t16[...] = cum
        tot = plsc.load_gather(t16, [jnp.full((16,), 15, jnp.int32)])
        return cnt + tot

    def chunk_body(c, carry):
        st = c * CHUNK
        pltpu.sync_copy(idx_hbm.at[b, pl.ds(st, CHUNK)], ibuf)

        def vbody(j, cnts):
            c0, c1 = cnts
            iv = ibuf[pl.ds(j * 16, 16)]
            pid = b * T + st + j * 16 + io
            rel = iv - lo
            m0 = (rel >= 0) & (rel < ESEG)
            m1 = (rel >= ESEG) & (rel < 2 * ESEG)
            c0 = emit(plb0, slb0, m0, rel, pid, c0)
            c1 = emit(plb1, slb1, m1, rel - ESEG, pid, c1)
            return (c0, c1)

        z16 = jnp.zeros((16,), jnp.int32)
        c0, c1 = lax.fori_loop(0, CHUNK // 16, vbody, (z16, z16))

        for p, (plb, slb, cv) in enumerate(((plb0, slb0, c0), (plb1, slb1, c1))):
            n = cv[0]
            npad = ((n + 63) // 64) * 64
            lastpos = jnp.maximum(n - 1, 0)
            lpl = plsc.load_gather(plb, [jnp.full((16,), lastpos, jnp.int32)])
            lsl = plsc.load_gather(slb, [jnp.full((16,), lastpos, jnp.int32)])
            for k in range(4):
                pos = n + k * 16 + io
                mm = pos < npad
                plsc.store_scatter(plb, [pos], lpl, mask=mm)
                plsc.store_scatter(slb, [pos], lsl, mask=mm)
            e = 2 * q + p
            pltpu.sync_copy(plb, plist.at[b, e, c])
            pltpu.sync_copy(slb, slist.at[b, e, c])
            plsc.store_scatter(cbuf, [jnp.full((16,), 2 * c + p, jnp.int32)],
                               jnp.full((16,), npad, jnp.int32), mask=(io == 0))
        return carry

    lax.fori_loop(0, NCH, chunk_body, 0)
    pltpu.sync_copy(cbuf, counts.at[b, q])


@functools.partial(
    pl.kernel,
    out_type=jax.ShapeDtypeStruct((ROWS, H), jnp.float32),
    mesh=_MESH,
    compiler_params=_SC,
    scratch_types=[
        pltpu.VMEM((ESEG * H,), jnp.float32),
        pltpu.VMEM((1, 64), jnp.int32),
        pltpu.VMEM((64,), jnp.int32),
        pltpu.VMEM((64, H), jnp.float32),
        pltpu.VMEM((64, H), jnp.float32),
        pltpu.VMEM((64,), jnp.int32),
        pltpu.SemaphoreType.DMA,
    ],
)
def _pool_kernel(netf, plist, slist, counts, pooled,
                 table, plb, slb, rows, orows, cbuf, sem):
    cc = lax.axis_index("c")
    ss = lax.axis_index("s")
    b = cc * 4 + ss // 4
    q = ss % 4
    io = lax.iota(jnp.int32, 16)

    pltpu.sync_copy(counts.at[b, q], cbuf)

    for p in (0, 1):
        e = 2 * q + p

        def initb(i, carry):
            table[pl.ds(i * 16, 16)] = jnp.full((16,), -jnp.inf, jnp.float32)
            return carry
        lax.fori_loop(0, ESEG * H // 16, initb, 0)

        def chunk_a(c, carry):
            nv = plsc.load_gather(cbuf, [jnp.full((16,), 2 * c + p, jnp.int32)])
            nblk = nv[0] // 64

            def blk_a(k, carry2):
                pltpu.sync_copy(plist.at[b, e, c, pl.ds(k * 64, 64)], plb.at[0])
                pltpu.sync_copy(slist.at[b, e, c, pl.ds(k * 64, 64)], slb)
                pltpu.async_copy(netf.at[plb.at[0]], rows, sem).wait()
                for g in range(4):
                    sv = slb[pl.ds(g * 16, 16)]
                    for jj in range(16):
                        s = sv[jj]
                        o = s * H
                        pt = jnp.full((16,), g * 16 + jj, jnp.int32)
                        rl = plsc.load_gather(rows, [pt, io])
                        rh = plsc.load_gather(rows, [pt, io + 16])
                        table[pl.ds(o, 16)] = jnp.maximum(table[pl.ds(o, 16)], rl)
                        table[pl.ds(o + 16, 16)] = jnp.maximum(table[pl.ds(o + 16, 16)], rh)
                return carry2
            lax.fori_loop(0, nblk, blk_a, 0)
            return carry
        lax.fori_loop(0, NCH, chunk_a, 0)

        def chunk_b(c, carry):
            nv = plsc.load_gather(cbuf, [jnp.full((16,), 2 * c + p, jnp.int32)])
            nblk = nv[0] // 64

            def blk_b(k, carry2):
                pltpu.sync_copy(plist.at[b, e, c, pl.ds(k * 64, 64)], plb.at[0])
                pltpu.sync_copy(slist.at[b, e, c, pl.ds(k * 64, 64)], slb)
                for g in range(4):
                    sv = slb[pl.ds(g * 16, 16)]
                    for jj in range(16):
                        s = sv[jj]
                        o = s * H
                        pt = jnp.full((16,), g * 16 + jj, jnp.int32)
                        plsc.store_scatter(orows, [pt, io], table[pl.ds(o, 16)])
                        plsc.store_scatter(orows, [pt, io + 16], table[pl.ds(o + 16, 16)])
                pltpu.sync_copy(orows, pooled.at[plb.at[0]])
                return carry2
            lax.fori_loop(0, nblk, blk_b, 0)
            return carry
        lax.fori_loop(0, NCH, chunk_b, 0)


@functools.partial(
    pl.kernel,
    out_type=jax.ShapeDtypeStruct((B * NS, CD), jnp.float32),
    mesh=_MESH,
    compiler_params=_SC,
    scratch_types=[
        pltpu.VMEM_SHARED((NS + 8, CD), jnp.float32),
        pltpu.VMEM_SHARED((NS + 8,), jnp.float32),
        pltpu.VMEM((128, CD), jnp.float32),
        pltpu.VMEM((1024,), jnp.float32),
        pltpu.VMEM((MCH,), jnp.float32),
        pltpu.VMEM((NMC + 1, MCH), jnp.int32),
        pltpu.VMEM((MCH, CD), jnp.float32),
        pltpu.VMEM((NS // 16, CD), jnp.float32),
        pltpu.VMEM((NS // 16,), jnp.float32),
        pltpu.SemaphoreType.DMA,
    ],
)
def _mean_kernel(cf, idxf, mean,
                 spm_s, spm_c, zbuf, zcnt, ones, ibx, rows, tb, cb, sem):
    cc = lax.axis_index("c")
    ss = lax.axis_index("s")
    io = lax.iota(jnp.int32, 16)
    pe = ss
    start = pe * MTS
    size = jnp.where(pe == 15, T - 15 * MTS, MTS)

    def zb(i, carry):
        plsc.store_scatter(zbuf, [jnp.full((16,), i, jnp.int32), io],
                           jnp.zeros((16,), jnp.float32))
        plsc.store_scatter(zbuf, [jnp.full((16,), i, jnp.int32), io + 16],
                           jnp.zeros((16,), jnp.float32))
        return carry
    lax.fori_loop(0, 128, zb, 0)

    def zc(i, carry):
        zcnt[pl.ds(i * 16, 16)] = jnp.zeros((16,), jnp.float32)
        return carry
    lax.fori_loop(0, 64, zc, 0)

    def ob(i, carry):
        ones[pl.ds(i * 16, 16)] = jnp.ones((16,), jnp.float32)
        return carry
    lax.fori_loop(0, MCH // 16, ob, 0)

    for hp in (0, 1, 2, 3):
        b = cc * 4 + hp

        def zs(i, carry):
            pltpu.sync_copy(zbuf, spm_s.at[pl.ds(ss * 1024 + i * 128, 128)])
            return carry
        lax.fori_loop(0, 8, zs, 0)
        pltpu.sync_copy(zcnt, spm_c.at[pl.ds(ss * 1024, 1024)])
        plsc.subcore_barrier()

        for ch in range(NMC + 1):
            if ch < NMC:
                stc = start + ch * MCH
                valid_from = None
            else:
                stc = start + size - MCH
                valid_from = start + NMC * MCH
            gst = b * T + stc
            pltpu.sync_copy(idxf.at[pl.ds(gst, MCH)], ibx.at[ch])
            pltpu.sync_copy(cf.at[pl.ds(gst, MCH)], rows)

            if valid_from is not None:
                def bias(j, carry):
                    chs = jnp.full((16,), ch, jnp.int32)
                    iv = plsc.load_gather(ibx, [chs, j * 16 + io])
                    gpos = stc + j * 16 + io
                    iv = jnp.where(gpos >= valid_from, iv, NS)
                    plsc.store_scatter(ibx, [chs, j * 16 + io], iv)
                    return carry
                lax.fori_loop(0, MCH // 16, bias, 0)

            pltpu.sync_copy(rows, spm_s.at[ibx.at[ch]], add=True)
            pltpu.sync_copy(ones, spm_c.at[ibx.at[ch]], add=True)
        plsc.subcore_barrier()

        srow = ss * (NS // 16)
        pltpu.sync_copy(spm_s.at[pl.ds(srow, NS // 16)], tb)
        pltpu.sync_copy(spm_c.at[pl.ds(srow, NS // 16)], cb)

        def dv(g, carry):
            cv = cb[pl.ds(g * 16, 16)]
            rec = 1.0 / jnp.maximum(cv, 1.0)
            for jj in range(16):
                r = rec[jj]
                o = (g * 16 + jj) * CD
                base = g * 16 + jj
                lo16 = plsc.load_gather(tb, [jnp.full((16,), base, jnp.int32), io])
                hi16 = plsc.load_gather(tb, [jnp.full((16,), base, jnp.int32), io + 16])
                plsc.store_scatter(tb, [jnp.full((16,), base, jnp.int32), io], lo16 * r)
                plsc.store_scatter(tb, [jnp.full((16,), base, jnp.int32), io + 16], hi16 * r)
            return carry
        lax.fori_loop(0, NS // 16 // 16, dv, 0)

        pltpu.sync_copy(tb, mean.at[pl.ds(b * NS + srow, NS // 16)])
        plsc.subcore_barrier()


def kernel(points, index, params):
    idx = index[:, 0, :].astype(jnp.int32)
    idxf = idx.reshape(ROWS)

    plist, slist, counts = _route_kernel(idx)

    net = _front(points, params)
    for i in range(1, NB - 1):
        pooled = _pool_kernel(net, plist, slist, counts)
        net = _mid(net, pooled, params['blocks'][i])
    pooled = _pool_kernel(net, plist, slist, counts)
    c = _last(net, pooled, params['blocks'][NB - 1], params['Wc'], params['bc'])

    mean = _mean_kernel(c, idxf)
    plane = _transpose_mean(mean)
    return plane.reshape(B, CD, RESO, RESO)


# speedup vs baseline: 5.6519x; 1.3615x over previous
import functools

import jax
import jax.numpy as jnp
from jax import lax
from jax.experimental import pallas as pl
from jax.experimental.pallas import tpu as pltpu
from jax.experimental.pallas import tpu_sc as plsc

B = 8
T = 50000
DIM = 3
H = 32
CD = 32
RESO = 128
NS = RESO * RESO
NB = 5

ROWS = B * T
BLK = 4000

NE = 8
ESEG = NS // NE
CHUNK = 2000
NCH = T // CHUNK
SLOT = CHUNK + 128
LCAP = NCH * SLOT

MTS = 3128
MCH = 512
NMC = 6

_SC = pltpu.CompilerParams(needs_layout_passes=False, use_tc_tiling_on_sc=False)
_MESH = plsc.VectorSubcoreMesh(core_axis_name="c", subcore_axis_name="s")


def _front_body(p_ref, wpos, bpos, w0, b0, w1, b1, ws, net_out):
    x = jnp.dot(p_ref[...], wpos[...], preferred_element_type=jnp.float32) + bpos[...]
    h = jnp.dot(jnp.maximum(x, 0.0), w0[...], preferred_element_type=jnp.float32) + b0[...]
    dx = jnp.dot(jnp.maximum(h, 0.0), w1[...], preferred_element_type=jnp.float32) + b1[...]
    net_out[...] = jnp.dot(x, ws[...], preferred_element_type=jnp.float32) + dx


def _mid_body(net_ref, pool_ref, w0, b0, w1, b1, ws, out_ref):
    x = jnp.concatenate([net_ref[...], pool_ref[...]], axis=-1)
    h = jnp.dot(jnp.maximum(x, 0.0), w0[...], preferred_element_type=jnp.float32) + b0[...]
    dx = jnp.dot(jnp.maximum(h, 0.0), w1[...], preferred_element_type=jnp.float32) + b1[...]
    out_ref[...] = jnp.dot(x, ws[...], preferred_element_type=jnp.float32) + dx


def _last_body(net_ref, pool_ref, w0, b0, w1, b1, ws, wc, bc, out_ref):
    x = jnp.concatenate([net_ref[...], pool_ref[...]], axis=-1)
    h = jnp.dot(jnp.maximum(x, 0.0), w0[...], preferred_element_type=jnp.float32) + b0[...]
    dx = jnp.dot(jnp.maximum(h, 0.0), w1[...], preferred_element_type=jnp.float32) + b1[...]
    net = jnp.dot(x, ws[...], preferred_element_type=jnp.float32) + dx
    out_ref[...] = jnp.dot(net, wc[...], preferred_element_type=jnp.float32) + bc[...]


def _tr_body(in_ref, out_ref):
    out_ref[...] = jnp.transpose(in_ref[...], (1, 0))[None]


def _row_spec(width):
    return pl.BlockSpec((BLK, width), lambda i: (i, 0))


def _full_spec(shape):
    nd = len(shape)
    return pl.BlockSpec(shape, lambda i: (0,) * nd)


def _front(points, params):
    bp = params['blocks'][0]
    args = (points.reshape(ROWS, DIM), params['Wpos'], params['bpos'].reshape(1, 2 * H),
            bp['W0'], bp['b0'].reshape(1, H), bp['W1'], bp['b1'].reshape(1, H), bp['Ws'])
    return pl.pallas_call(
        _front_body,
        grid=(ROWS // BLK,),
        in_specs=[_row_spec(DIM)] + [_full_spec(a.shape) for a in args[1:]],
        out_specs=_row_spec(H),
        out_shape=jax.ShapeDtypeStruct((ROWS, H), jnp.float32),
    )(*args)


def _mid(net, pooled, bp):
    args = (net, pooled, bp['W0'], bp['b0'].reshape(1, H), bp['W1'],
            bp['b1'].reshape(1, H), bp['Ws'])
    return pl.pallas_call(
        _mid_body,
        grid=(ROWS // BLK,),
        in_specs=[_row_spec(H), _row_spec(H)] + [_full_spec(a.shape) for a in args[2:]],
        out_specs=_row_spec(H),
        out_shape=jax.ShapeDtypeStruct((ROWS, H), jnp.float32),
    )(*args)


def _last(net, pooled, bp, wc, bc):
    args = (net, pooled, bp['W0'], bp['b0'].reshape(1, H), bp['W1'],
            bp['b1'].reshape(1, H), bp['Ws'], wc, bc.reshape(1, CD))
    return pl.pallas_call(
        _last_body,
        grid=(ROWS // BLK,),
        in_specs=[_row_spec(H), _row_spec(H)] + [_full_spec(a.shape) for a in args[2:]],
        out_specs=_row_spec(CD),
        out_shape=jax.ShapeDtypeStruct((ROWS, CD), jnp.float32),
    )(*args)


def _transpose_mean(mean_flat):
    return pl.pallas_call(
        _tr_body,
        grid=(B, NS // 2048),
        in_specs=[pl.BlockSpec((2048, CD), lambda b, t: (b * (NS // 2048) + t, 0))],
        out_specs=pl.BlockSpec((1, CD, 2048), lambda b, t: (b, 0, t)),
        out_shape=jax.ShapeDtypeStruct((B, CD, NS), jnp.float32),
    )(mean_flat)


def _vsplat(v, lane):
    dn = lax.GatherDimensionNumbers(offset_dims=(), collapsed_slice_dims=(0,),
                                    start_index_map=(0,))
    return lax.gather(v, jnp.full((16, 1), lane, jnp.int32), dn, slice_sizes=(1,),
                      mode=lax.GatherScatterMode.PROMISE_IN_BOUNDS)


@functools.partial(
    pl.kernel,
    out_type=[
        jax.ShapeDtypeStruct((B, NE, LCAP), jnp.int32),
        jax.ShapeDtypeStruct((B, NE, LCAP), jnp.int32),
        jax.ShapeDtypeStruct((B, 4, 64), jnp.int32),
    ],
    mesh=_MESH,
    compiler_params=_SC,
    scratch_types=[
        pltpu.VMEM((CHUNK,), jnp.int32),
        pltpu.VMEM((SLOT,), jnp.int32),
        pltpu.VMEM((SLOT,), jnp.int32),
        pltpu.VMEM((SLOT,), jnp.int32),
        pltpu.VMEM((SLOT,), jnp.int32),
        pltpu.VMEM((64,), jnp.int32),
        pltpu.VMEM((16,), jnp.int32),
    ],
)
def _route_kernel(idx_hbm, plist, slist, counts,
                  ibuf, plb0, slb0, plb1, slb1, cbuf, t16):
    cc = lax.axis_index("c")
    ss = lax.axis_index("s")
    b = cc * 4 + ss // 4
    q = ss % 4
    lo = q * (2 * ESEG)
    io = lax.iota(jnp.int32, 16)

    def zb(i, carry):
        cbuf[pl.ds(i * 16, 16)] = jnp.zeros((16,), jnp.int32)
        return carry
    lax.fori_loop(0, 4, zb, 0)

    def emit(plb, slb, m, sbias, pid, cnt):
        cum = m.astype(jnp.int32)
        for d in (1, 2, 4, 8):
            t16[...] = cum
            sh = plsc.load_gather(t16, [jnp.maximum(io - d, 0)])
            cum = cum + jnp.where(io >= d, sh, 0)
        pos = cnt + cum - 1
        plsc.store_scatter(plb, [pos], pid, mask=m)
        plsc.store_scatter(slb, [pos], sbias, mask=m)
        t16[...] = cum
        tot = plsc.load_gather(t16, [jnp.full((16,), 15, jnp.int32)])
        return cnt + tot

    def chunk_body(c, bases):
        base0, base1 = bases
        st = c * CHUNK
        pltpu.sync_copy(idx_hbm.at[b, pl.ds(st, CHUNK)], ibuf)

        def vbody(j, cnts):
            c0, c1 = cnts
            iv = ibuf[pl.ds(j * 16, 16)]
            pid = b * T + st + j * 16 + io
            sbias = iv + b * NS
            rel = iv - lo
            m0 = (rel >= 0) & (rel < ESEG)
            m1 = (rel >= ESEG) & (rel < 2 * ESEG)
            c0 = emit(plb0, slb0, m0, sbias, pid, c0)
            c1 = emit(plb1, slb1, m1, sbias, pid, c1)
            return (c0, c1)

        z16 = jnp.zeros((16,), jnp.int32)
        c0, c1 = lax.fori_loop(0, CHUNK // 16, vbody, (z16, z16))

        newbases = []
        for p, (plb, slb, cv, basev) in enumerate(
                ((plb0, slb0, c0, base0), (plb1, slb1, c1, base1))):
            n = cv[0]
            npad = ((n + 127) // 128) * 128
            lastpos = jnp.maximum(n - 1, 0)
            lpl = plsc.load_gather(plb, [jnp.full((16,), lastpos, jnp.int32)])
            lsl = plsc.load_gather(slb, [jnp.full((16,), lastpos, jnp.int32)])
            for k in range(8):
                pos = n + k * 16 + io
                mm = pos < npad
                plsc.store_scatter(plb, [pos], lpl, mask=mm)
                plsc.store_scatter(slb, [pos], lsl, mask=mm)
            e = 2 * q + p
            base = pl.multiple_of(basev[0], 128)
            pltpu.sync_copy(plb, plist.at[b, e, pl.ds(base, SLOT)])
            pltpu.sync_copy(slb, slist.at[b, e, pl.ds(base, SLOT)])
            newbases.append(basev + npad)
        return tuple(newbases)

    z16b = jnp.zeros((16,), jnp.int32)
    t0, t1 = lax.fori_loop(0, NCH, chunk_body, (z16b, z16b))
    plsc.store_scatter(cbuf, [jnp.full((16,), 50, jnp.int32)], t0, mask=(io == 0))
    plsc.store_scatter(cbuf, [jnp.full((16,), 51, jnp.int32)], t1, mask=(io == 0))
    pltpu.sync_copy(cbuf, counts.at[b, q])


@functools.partial(
    pl.kernel,
    out_type=jax.ShapeDtypeStruct((ROWS, H), jnp.float32),
    mesh=_MESH,
    compiler_params=_SC,
    scratch_types=[
        pltpu.VMEM((ESEG * H,), jnp.float32),
        pltpu.VMEM((2, 128), jnp.int32),
        pltpu.VMEM((2, 128), jnp.int32),
        pltpu.VMEM((2, 128, H), jnp.float32),
        pltpu.VMEM((2, 128), jnp.int32),
        pltpu.VMEM((2, 128, H), jnp.float32),
        pltpu.VMEM((64,), jnp.int32),
        pltpu.SemaphoreType.DMA,
        pltpu.SemaphoreType.DMA,
        pltpu.SemaphoreType.DMA,
        pltpu.SemaphoreType.DMA,
        pltpu.SemaphoreType.DMA,
        pltpu.SemaphoreType.DMA,
    ],
)
def _pool_kernel(netf, plist, slist, counts, pooled,
                 table, plb, slb, rows, pidx, orows, cbuf,
                 lp0, ls0, lp1, ls1, g0, g1):
    cc = lax.axis_index("c")
    ss = lax.axis_index("s")
    b = cc * 4 + ss // 4
    q = ss % 4
    io = lax.iota(jnp.int32, 16)

    pltpu.sync_copy(counts.at[b, q], cbuf)
    lsem = (lp0, lp1)
    ssem = (ls0, ls1)
    gsem = (g0, g1)

    for p in (0, 1):
        e = 2 * q + p
        base = b * NS + e * ESEG
        tv = plsc.load_gather(cbuf, [jnp.full((16,), 50 + p, jnp.int32)])
        total = pl.multiple_of(tv[0], 128)
        nblk = total // 128

        def initb(i, carry):
            table[pl.ds(i * 16, 16)] = jnp.full((16,), -jnp.inf, jnp.float32)
            return carry
        lax.fori_loop(0, ESEG * H // 16, initb, 0)

        @pl.when(nblk > 0)
        def _():
            pltpu.async_copy(plist.at[b, e, pl.ds(0, 128)], plb.at[0], lp0)
            pltpu.async_copy(slist.at[b, e, pl.ds(0, 128)], slb.at[0], ls0)

        def pair_a(kp, carry):
            for sl in (0, 1):
                k = 2 * kp + sl

                @pl.when(k < nblk)
                def _():
                    pltpu.make_async_copy(plist.at[b, e, pl.ds(0, 128)],
                                          plb.at[sl], lsem[sl]).wait()
                    pltpu.make_async_copy(slist.at[b, e, pl.ds(0, 128)],
                                          slb.at[sl], ssem[sl]).wait()
                    pltpu.async_copy(netf.at[plb.at[sl]], rows.at[sl], gsem[sl])

                @pl.when((k >= 1) & (k <= nblk))
                def _():
                    osl = 1 - sl
                    pltpu.make_async_copy(netf.at[plb.at[osl]],
                                          rows.at[osl], gsem[osl]).wait()

                    def grp(g, c2):
                        sv = slb.at[osl][pl.ds(g * 16, 16)]
                        for jj in range(16):
                            o = (sv[jj] - base) * H
                            pt = jnp.full((16,), g * 16 + jj, jnp.int32)
                            rl = plsc.load_gather(rows.at[osl], [pt, io])
                            rh = plsc.load_gather(rows.at[osl], [pt, io + 16])
                            table[pl.ds(o, 16)] = jnp.maximum(table[pl.ds(o, 16)], rl)
                            table[pl.ds(o + 16, 16)] = jnp.maximum(
                                table[pl.ds(o + 16, 16)], rh)
                        return c2
                    lax.fori_loop(0, 8, grp, 0)

                @pl.when(k + 1 < nblk)
                def _():
                    nsl = 1 - sl
                    pltpu.async_copy(plist.at[b, e, pl.ds((k + 1) * 128, 128)],
                                     plb.at[nsl], lsem[nsl])
                    pltpu.async_copy(slist.at[b, e, pl.ds((k + 1) * 128, 128)],
                                     slb.at[nsl], ssem[nsl])
            return carry
        lax.fori_loop(0, (nblk + 2) // 2, pair_a, 0)

        @pl.when(nblk > 0)
        def _():
            pltpu.async_copy(plist.at[b, e, pl.ds(0, 128)], plb.at[0], lp0)
            pltpu.async_copy(slist.at[b, e, pl.ds(0, 128)], slb.at[0], ls0)

        def pair_b(kp, carry):
            for sl in (0, 1):
                k = 2 * kp + sl

                @pl.when(k < nblk)
                def _():
                    pltpu.make_async_copy(plist.at[b, e, pl.ds(0, 128)],
                                          plb.at[sl], lsem[sl]).wait()
                    pltpu.make_async_copy(slist.at[b, e, pl.ds(0, 128)],
                                          slb.at[sl], ssem[sl]).wait()

                    @pl.when(k >= 2)
                    def _():
                        pltpu.make_async_copy(orows.at[sl],
                                              pooled.at[pidx.at[sl]], gsem[sl]).wait()

                    def grp(g, c2):
                        sv = slb.at[sl][pl.ds(g * 16, 16)]
                        for jj in range(16):
                            o = (sv[jj] - base) * H
                            pt = jnp.full((16,), g * 16 + jj, jnp.int32)
                            plsc.store_scatter(orows.at[sl], [pt, io],
                                               table[pl.ds(o, 16)])
                            plsc.store_scatter(orows.at[sl], [pt, io + 16],
                                               table[pl.ds(o + 16, 16)])
                        return c2
                    lax.fori_loop(0, 8, grp, 0)

                    for g2 in range(8):
                        pidx.at[sl][pl.ds(g2 * 16, 16)] = (
                            plb.at[sl][pl.ds(g2 * 16, 16)])
                    pltpu.async_copy(orows.at[sl], pooled.at[pidx.at[sl]], gsem[sl])

                @pl.when(k + 1 < nblk)
                def _():
                    nsl = 1 - sl
                    pltpu.async_copy(plist.at[b, e, pl.ds((k + 1) * 128, 128)],
                                     plb.at[nsl], lsem[nsl])
                    pltpu.async_copy(slist.at[b, e, pl.ds((k + 1) * 128, 128)],
                                     slb.at[nsl], ssem[nsl])
            return carry
        lax.fori_loop(0, (nblk + 1) // 2, pair_b, 0)

        @pl.when(nblk >= 1)
        def _():
            for slp in (0, 1):
                @pl.when((nblk - 1) % 2 == slp)
                def _():
                    pltpu.make_async_copy(orows.at[slp], pooled.at[pidx.at[slp]],
                                          gsem[slp]).wait()

        @pl.when(nblk >= 2)
        def _():
            for slp in (0, 1):
                @pl.when((nblk - 2) % 2 == slp)
                def _():
                    pltpu.make_async_copy(orows.at[slp], pooled.at[pidx.at[slp]],
                                          gsem[slp]).wait()


@functools.partial(
    pl.kernel,
    out_type=jax.ShapeDtypeStruct((B * NS, CD), jnp.float32),
    mesh=_MESH,
    compiler_params=_SC,
    scratch_types=[
        pltpu.VMEM_SHARED((NS + 8, CD), jnp.float32),
        pltpu.VMEM_SHARED((NS + 8,), jnp.float32),
        pltpu.VMEM((128, CD), jnp.float32),
        pltpu.VMEM((1024,), jnp.float32),
        pltpu.VMEM((MCH,), jnp.float32),
        pltpu.VMEM((NMC + 1, MCH), jnp.int32),
        pltpu.VMEM((MCH, CD), jnp.float32),
        pltpu.VMEM((NS // 16, CD), jnp.float32),
        pltpu.VMEM((NS // 16,), jnp.float32),
        pltpu.SemaphoreType.DMA,
    ],
)
def _mean_kernel(cf, idxf, mean,
                 spm_s, spm_c, zbuf, zcnt, ones, ibx, rows, tb, cb, sem):
    cc = lax.axis_index("c")
    ss = lax.axis_index("s")
    io = lax.iota(jnp.int32, 16)
    pe = ss
    start = pe * MTS
    size = jnp.where(pe == 15, T - 15 * MTS, MTS)

    def zb(i, carry):
        plsc.store_scatter(zbuf, [jnp.full((16,), i, jnp.int32), io],
                           jnp.zeros((16,), jnp.float32))
        plsc.store_scatter(zbuf, [jnp.full((16,), i, jnp.int32), io + 16],
                           jnp.zeros((16,), jnp.float32))
        return carry
    lax.fori_loop(0, 128, zb, 0)

    def zc(i, carry):
        zcnt[pl.ds(i * 16, 16)] = jnp.zeros((16,), jnp.float32)
        return carry
    lax.fori_loop(0, 64, zc, 0)

    def ob(i, carry):
        ones[pl.ds(i * 16, 16)] = jnp.ones((16,), jnp.float32)
        return carry
    lax.fori_loop(0, MCH // 16, ob, 0)

    for hp in (0, 1, 2, 3):
        b = cc * 4 + hp

        def zs(i, carry):
            pltpu.sync_copy(zbuf, spm_s.at[pl.ds(ss * 1024 + i * 128, 128)])
            return carry
        lax.fori_loop(0, 8, zs, 0)
        pltpu.sync_copy(zcnt, spm_c.at[pl.ds(ss * 1024, 1024)])
        plsc.subcore_barrier()

        for ch in range(NMC + 1):
            if ch < NMC:
                stc = start + ch * MCH
                valid_from = None
            else:
                stc = start + size - MCH
                valid_from = start + NMC * MCH
            gst = b * T + stc
            pltpu.sync_copy(idxf.at[pl.ds(gst, MCH)], ibx.at[ch])
            pltpu.sync_copy(cf.at[pl.ds(gst, MCH)], rows)

            if valid_from is not None:
                def bias(j, carry):
                    chs = jnp.full((16,), ch, jnp.int32)
                    iv = plsc.load_gather(ibx, [chs, j * 16 + io])
                    gpos = stc + j * 16 + io
                    iv = jnp.where(gpos >= valid_from, iv, NS)
                    plsc.store_scatter(ibx, [chs, j * 16 + io], iv)
                    return carry
                lax.fori_loop(0, MCH // 16, bias, 0)

            pltpu.sync_copy(rows, spm_s.at[ibx.at[ch]], add=True)
            pltpu.sync_copy(ones, spm_c.at[ibx.at[ch]], add=True)
        plsc.subcore_barrier()

        srow = ss * (NS // 16)
        pltpu.sync_copy(spm_s.at[pl.ds(srow, NS // 16)], tb)
        pltpu.sync_copy(spm_c.at[pl.ds(srow, NS // 16)], cb)

        def dv(g, carry):
            cv = cb[pl.ds(g * 16, 16)]
            rec = 1.0 / jnp.maximum(cv, 1.0)
            for jj in range(16):
                r = rec[jj]
                o = (g * 16 + jj) * CD
                base = g * 16 + jj
                lo16 = plsc.load_gather(tb, [jnp.full((16,), base, jnp.int32), io])
                hi16 = plsc.load_gather(tb, [jnp.full((16,), base, jnp.int32), io + 16])
                plsc.store_scatter(tb, [jnp.full((16,), base, jnp.int32), io], lo16 * r)
                plsc.store_scatter(tb, [jnp.full((16,), base, jnp.int32), io + 16], hi16 * r)
            return carry
        lax.fori_loop(0, NS // 16 // 16, dv, 0)

        pltpu.sync_copy(tb, mean.at[pl.ds(b * NS + srow, NS // 16)])
        plsc.subcore_barrier()


def kernel(points, index, params):
    idx = index[:, 0, :].astype(jnp.int32)
    idxf = idx.reshape(ROWS)

    plist, slist, counts = _route_kernel(idx)

    net = _front(points, params)
    for i in range(1, NB - 1):
        pooled = _pool_kernel(net, plist, slist, counts)
        net = _mid(net, pooled, params['blocks'][i])
    pooled = _pool_kernel(net, plist, slist, counts)
    c = _last(net, pooled, params['blocks'][NB - 1], params['Wc'], params['bc'])

    mean = _mean_kernel(c, idxf)
    plane = _transpose_mean(mean)
    return plane.reshape(B, CD, RESO, RESO)
